# Initial kernel scaffold; baseline (speedup 1.0000x reference)
import jax
import jax.numpy as jnp
from jax.experimental import pallas as pl


def kernel(x, buckets, W1, b1, a1, W2, b2, a2, emb):
    raise NotImplementedError("write your pallas kernel here")


# speedup vs baseline: 6.1197x; 6.1197x over previous
import functools

import jax
import jax.numpy as jnp
from jax import lax
from jax.experimental import pallas as pl
from jax.experimental.pallas import tpu as pltpu
from jax.experimental.pallas import tpu_sc as plsc

HH = 384
WW = 384
CC = 32
TT = 216
NN = HH * WW
CP = 640
CO = 8
RB = 12
NB = HH // RB
MT = 512
NT = NN // MT
NW = 32
CHK = 128
N9 = NN * 9
C9W = N9 // NW // CHK
C1W = NN // NW // CHK


def _shift(v, s):
    if s == 0:
        return v
    return pltpu.roll(v, (-s) % v.shape[1], 1)


def _k1_body(xq_hbm, w1_ref, w2_ref, aa_ref, bb_ref, out_ref, xp_slab, sem):
    i = pl.program_id(0)
    cp = pltpu.make_async_copy(
        xq_hbm.at[:, pl.ds((i * RB + 1) * CP, (RB + 6) * CP)], xp_slab, sem)
    cp.start()
    cp.wait()
    xp = xp_slab[...]
    a1 = aa_ref[0]
    a2 = aa_ref[1]

    l1 = (RB + 2) * CP
    y1 = jnp.zeros((CC, l1), jnp.float32)
    for b in range(3):
        xs = _shift(xp, b - 1)
        for a in range(3):
            y1 = y1 + jnp.dot(w1_ref[3 * a + b], xs[:, (a + 1) * CP:(a + 1) * CP + l1],
                              preferred_element_type=jnp.float32)
    y1 = y1 + bb_ref[0][:, None]
    y1 = jnp.where(y1 >= 0, y1, a1 * y1)
    idx1 = lax.broadcasted_iota(jnp.int32, (CC, l1), 1)
    col1 = idx1 % CP
    q = i * RB + 3 + idx1 // CP
    ok1 = (col1 >= CO) & (col1 < CO + WW) & (q >= 4) & (q <= 387)
    y1 = jnp.where(ok1, y1, 0.0)

    l2 = RB * CP
    out = jnp.zeros((l2, CC), jnp.float32)
    for b in range(3):
        ys = _shift(y1, b - 1)
        for a in range(3):
            out = out + lax.dot_general(
                ys[:, a * CP:a * CP + l2], w2_ref[3 * a + b],
                ((( 0,), (0,)), ((), ())), preferred_element_type=jnp.float32)
    out = out + bb_ref[1][None, :]
    out = jnp.where(out >= 0, out, a2 * out)
    col2 = lax.broadcasted_iota(jnp.int32, (l2, CC), 0) % CP
    ok2 = (col2 >= CO) & (col2 < CO + WW)
    out_ref[...] = jnp.where(ok2, out, 0.0)


def _k1(xq, w1m, w2m, aa, bb):
    return pl.pallas_call(
        _k1_body,
        grid=(NB,),
        in_specs=[
            pl.BlockSpec(memory_space=pl.ANY),
            pl.BlockSpec((9, CC, CC), lambda i: (0, 0, 0)),
            pl.BlockSpec((9, CC, CC), lambda i: (0, 0, 0)),
            pl.BlockSpec(memory_space=pltpu.SMEM),
            pl.BlockSpec((2, CC), lambda i: (0, 0)),
        ],
        out_specs=pl.BlockSpec((RB * CP, CC), lambda i: (i, 0)),
        out_shape=jax.ShapeDtypeStruct((HH * CP, CC), jnp.float32),
        scratch_shapes=[pltpu.VMEM((CC, (RB + 6) * CP), jnp.float32),
                        pltpu.SemaphoreType.DMA],
    )(xq, w1m, w2m, aa, bb)


def _k2(table, xhwc, idx9, sidx):
    mesh = plsc.VectorSubcoreMesh(core_axis_name="c", subcore_axis_name="s")

    @functools.partial(
        pl.kernel,
        out_type=[jax.ShapeDtypeStruct((N9, CC), jnp.float32),
                  jax.ShapeDtypeStruct((NN, CC), jnp.float32)],
        mesh=mesh,
        scratch_types=[pltpu.VMEM((C9W, CHK), jnp.int32),
                       pltpu.VMEM((C1W, CHK), jnp.int32),
                       pltpu.VMEM((CHK, CC), jnp.float32),
                       pltpu.SemaphoreType.DMA],
        compiler_params=pltpu.CompilerParams(use_tc_tiling_on_sc=False),
    )
    def body(table_hbm, xhwc_hbm, idx9_hbm, sidx_hbm, p_out, xs_out,
             idx_v, sidx_v, rows_v, sem):
        wid = lax.axis_index("s") * 2 + lax.axis_index("c")
        pltpu.sync_copy(idx9_hbm.at[wid], idx_v)

        def g9(j, carry):
            pltpu.async_copy(table_hbm.at[idx_v.at[j]], rows_v, sem).wait()
            pltpu.sync_copy(rows_v, p_out.at[pl.ds((wid * C9W + j) * CHK, CHK)])
            return carry

        lax.fori_loop(0, C9W, g9, 0)
        pltpu.sync_copy(sidx_hbm.at[wid], sidx_v)

        def g1(j, carry):
            pltpu.async_copy(xhwc_hbm.at[sidx_v.at[j]], rows_v, sem).wait()
            pltpu.sync_copy(rows_v, xs_out.at[pl.ds((wid * C1W + j) * CHK, CHK)])
            return carry

        lax.fori_loop(0, C1W, g1, 0)

    return body(table, xhwc, idx9, sidx)


def _k3_body(s_ref, e_ref, b_ref, p_ref, xs_ref, wk_ref, bias_ref, out_ref):
    i = pl.program_id(0)
    s = s_ref[i]
    e = e_ref[i]
    p = p_ref[...]
    xs = xs_ref[...]
    bt = b_ref[0]

    def body(t, acc):
        w = wk_ref[t]
        y = jnp.dot(p, w, preferred_element_type=jnp.float32)
        y = y + bias_ref[t] + xs
        return jnp.where(bt == t, y, acc)

    out_ref[...] = lax.fori_loop(s, e + 1, body, jnp.zeros((MT, CC), jnp.float32))


def _k3(p_mat, xs, bsor3, wk, bias, s_arr, e_arr):
    grid_spec = pltpu.PrefetchScalarGridSpec(
        num_scalar_prefetch=2,
        grid=(NT,),
        in_specs=[
            pl.BlockSpec((1, MT, 1), lambda i, s, e: (i, 0, 0)),
            pl.BlockSpec((MT, 9 * CC), lambda i, s, e: (i, 0)),
            pl.BlockSpec((MT, CC), lambda i, s, e: (i, 0)),
            pl.BlockSpec((TT, 9 * CC, CC), lambda i, s, e: (0, 0, 0)),
            pl.BlockSpec((TT, 1, CC), lambda i, s, e: (0, 0, 0)),
        ],
        out_specs=pl.BlockSpec((MT, CC), lambda i, s, e: (i, 0)),
    )
    return pl.pallas_call(
        _k3_body,
        grid_spec=grid_spec,
        out_shape=jax.ShapeDtypeStruct((NN, CC), jnp.float32),
    )(s_arr, e_arr, bsor3, p_mat, xs, wk, bias)


def _k4(osort, sidx):
    mesh = plsc.VectorSubcoreMesh(core_axis_name="c", subcore_axis_name="s")

    @functools.partial(
        pl.kernel,
        out_type=jax.ShapeDtypeStruct((NN, CC), jnp.float32),
        mesh=mesh,
        scratch_types=[pltpu.VMEM((C1W, CHK), jnp.int32),
                       pltpu.VMEM((CHK, CC), jnp.float32),
                       pltpu.SemaphoreType.DMA],
        compiler_params=pltpu.CompilerParams(use_tc_tiling_on_sc=False),
    )
    def body(osort_hbm, sidx_hbm, out_hbm, sidx_v, buf_v, sem):
        wid = lax.axis_index("s") * 2 + lax.axis_index("c")
        pltpu.sync_copy(sidx_hbm.at[wid], sidx_v)

        def sc(j, carry):
            pltpu.sync_copy(osort_hbm.at[pl.ds((wid * C1W + j) * CHK, CHK)], buf_v)
            pltpu.async_copy(buf_v, out_hbm.at[sidx_v.at[j]], sem).wait()
            return carry

        lax.fori_loop(0, C1W, sc, 0)

    return body(osort, sidx)


def kernel(x, buckets, W1, b1, a1, W2, b2, a2, emb):
    f32 = jnp.float32
    bflat = buckets.reshape(NN).astype(jnp.int32)
    b_sorted, sort_idx = lax.sort_key_val(bflat, lax.iota(jnp.int32, NN))
    s_arr = b_sorted[0::MT]
    e_arr = b_sorted[MT - 1::MT]
    bsor3 = b_sorted.reshape(NT, MT, 1)

    h = sort_idx // WW
    w = sort_idx % WW
    offs = []
    for dy in (-1, 0, 1):
        rr = h + dy
        okr = (rr >= 0) & (rr < HH)
        for dx in (-1, 0, 1):
            offs.append(jnp.where(okr, rr * CP + (w + dx) + CO, 0))
    idx9 = jnp.stack(offs, axis=1).reshape(NW, C9W, CHK)
    sidx = sort_idx.reshape(NW, C1W, CHK)

    w1m = W1.reshape(CC, CC, 9).transpose(2, 0, 1)
    w2m = W2.reshape(CC, CC, 9).transpose(2, 1, 0)
    aa = jnp.stack([a1[0], a2[0]])
    wk = emb[:, :CC * CC * 9].reshape(TT, CC, CC, 3, 3)
    wk = (0.1 * wk.transpose(0, 3, 4, 2, 1)).reshape(TT, 9 * CC, CC)
    bias = (0.1 * emb[:, CC * CC * 9:]).reshape(TT, 1, CC)
    bb = jnp.stack([b1, b2])

    xq = jnp.pad(x[0], ((0, 0), (4, 4), (CO, CP - CO - WW)))
    xq = xq.reshape(CC, 392 * CP)
    table = _k1(xq, w1m, w2m, aa, bb)

    xhwc = x.reshape(CC, NN).T

    p_mat, xs = _k2(table, xhwc, idx9, sidx)
    p_mat = p_mat.reshape(NN, 9 * CC)

    osort = _k3(p_mat, xs, bsor3, wk, bias, s_arr, e_arr)

    out_hwc = _k4(osort, sidx)
    return out_hwc.T.reshape(1, CC, HH, WW)


# speedup vs baseline: 7.4003x; 1.2093x over previous
import functools

import jax
import jax.numpy as jnp
from jax import lax
from jax.experimental import pallas as pl
from jax.experimental.pallas import tpu as pltpu
from jax.experimental.pallas import tpu_sc as plsc

HH = 384
WW = 384
CC = 32
TT = 216
NN = HH * WW
CP = 640
CO = 8
RB = 12
NB = HH // RB
MT = 512
NT = NN // MT
NW = 32
CHK = 128
GR = 6
N9 = NN * 9
C9W = N9 // NW // CHK
C1W = NN // NW // CHK
G9 = C9W // GR
G1 = C1W // GR


def _shift(v, s):
    if s == 0:
        return v
    return pltpu.roll(v, (-s) % v.shape[1], 1)


def _k1_copy(xq_hbm, xp_slab, sem, i, ph):
    return pltpu.make_async_copy(
        xq_hbm.at[:, pl.ds((i * RB + 1) * CP, (RB + 6) * CP)],
        xp_slab.at[ph], sem.at[ph])


def _k1_body(xq_hbm, w1_ref, w2_ref, aa_ref, bb_ref, out_ref, xp_slab, sem):
    i = pl.program_id(0)
    ph = lax.rem(i, 2)
    nph = lax.rem(i + 1, 2)

    @pl.when(i == 0)
    def _():
        _k1_copy(xq_hbm, xp_slab, sem, 0, 0).start()

    @pl.when(i + 1 < NB)
    def _():
        _k1_copy(xq_hbm, xp_slab, sem, i + 1, nph).start()

    _k1_copy(xq_hbm, xp_slab, sem, i, ph).wait()
    xp = xp_slab[ph].astype(jnp.bfloat16)
    a1 = aa_ref[0]
    a2 = aa_ref[1]

    l1 = (RB + 2) * CP
    y1 = jnp.zeros((CC, l1), jnp.float32)
    for b in range(3):
        xs = _shift(xp, b - 1)
        for a in range(3):
            y1 = y1 + jnp.dot(w1_ref[3 * a + b], xs[:, (a + 1) * CP:(a + 1) * CP + l1],
                              preferred_element_type=jnp.float32)
    y1 = y1 + bb_ref[0][:, None]
    y1 = jnp.where(y1 >= 0, y1, a1 * y1)
    idx1 = lax.broadcasted_iota(jnp.int32, (CC, l1), 1)
    col1 = idx1 % CP
    q = i * RB + 3 + idx1 // CP
    ok1 = (col1 >= CO) & (col1 < CO + WW) & (q >= 4) & (q <= 387)
    y1 = jnp.where(ok1, y1, 0.0)

    y1 = y1.astype(jnp.bfloat16)
    l2 = RB * CP
    out = jnp.zeros((l2, CC), jnp.float32)
    for b in range(3):
        ys = _shift(y1, b - 1)
        for a in range(3):
            out = out + lax.dot_general(
                ys[:, a * CP:a * CP + l2], w2_ref[3 * a + b],
                ((( 0,), (0,)), ((), ())), preferred_element_type=jnp.float32)
    out = out + bb_ref[1][None, :]
    out = jnp.where(out >= 0, out, a2 * out)
    col2 = lax.broadcasted_iota(jnp.int32, (l2, CC), 0) % CP
    ok2 = (col2 >= CO) & (col2 < CO + WW)
    out_ref[...] = jnp.where(ok2, out, 0.0)


def _k1(xq, w1m, w2m, aa, bb):
    return pl.pallas_call(
        _k1_body,
        grid=(NB,),
        in_specs=[
            pl.BlockSpec(memory_space=pl.ANY),
            pl.BlockSpec((9, CC, CC), lambda i: (0, 0, 0)),
            pl.BlockSpec((9, CC, CC), lambda i: (0, 0, 0)),
            pl.BlockSpec(memory_space=pltpu.SMEM),
            pl.BlockSpec((2, CC), lambda i: (0, 0)),
        ],
        out_specs=pl.BlockSpec((RB * CP, CC), lambda i: (i, 0)),
        out_shape=jax.ShapeDtypeStruct((HH * CP, CC), jnp.float32),
        scratch_shapes=[pltpu.VMEM((2, CC, (RB + 6) * CP), jnp.float32),
                        pltpu.SemaphoreType.DMA((2,))],
    )(xq, w1m, w2m, aa, bb)


def _k2(table, xhwc, idx9, sidx):
    mesh = plsc.VectorSubcoreMesh(core_axis_name="c", subcore_axis_name="s")

    @functools.partial(
        pl.kernel,
        out_type=[jax.ShapeDtypeStruct((N9, CC), jnp.float32),
                  jax.ShapeDtypeStruct((NN, CC), jnp.float32)],
        mesh=mesh,
        scratch_types=[pltpu.VMEM((C9W * CHK,), jnp.int32),
                       pltpu.VMEM((C1W * CHK,), jnp.int32),
                       pltpu.VMEM((GR * CHK, CC), jnp.float32),
                       pltpu.VMEM((GR * CHK, CC), jnp.float32),
                       pltpu.SemaphoreType.DMA,
                       pltpu.SemaphoreType.DMA],
        compiler_params=pltpu.CompilerParams(use_tc_tiling_on_sc=False),
    )
    def body(table_hbm, xhwc_hbm, idx9_hbm, sidx_hbm, p_out, xs_out,
             idx_v, sidx_v, rows_a, rows_b, sem_a, sem_b):
        wid = lax.axis_index("s") * 2 + lax.axis_index("c")
        pltpu.sync_copy(idx9_hbm.at[wid], idx_v)

        def gather(src_hbm, iv, dst_hbm, base, j, buf, sem, start_only):
            cp = pltpu.make_async_copy(
                src_hbm.at[iv.at[pl.ds(j * GR * CHK, GR * CHK)]], buf, sem)
            if start_only:
                cp.start()
            else:
                cp.wait()
                pltpu.sync_copy(
                    buf, dst_hbm.at[pl.ds((base + j * GR) * CHK, GR * CHK)])

        gather(table_hbm, idx_v, p_out, wid * C9W, 0, rows_a, sem_a, True)

        def g9(j2, carry):
            j = j2 * 2

            @pl.when(j + 1 < G9)
            def _():
                gather(table_hbm, idx_v, p_out, wid * C9W, j + 1,
                       rows_b, sem_b, True)

            gather(table_hbm, idx_v, p_out, wid * C9W, j, rows_a, sem_a, False)

            @pl.when(j + 2 < G9)
            def _():
                gather(table_hbm, idx_v, p_out, wid * C9W, j + 2,
                       rows_a, sem_a, True)

            @pl.when(j + 1 < G9)
            def _():
                gather(table_hbm, idx_v, p_out, wid * C9W, j + 1,
                       rows_b, sem_b, False)

            return carry

        lax.fori_loop(0, (G9 + 1) // 2, g9, 0)
        pltpu.sync_copy(sidx_hbm.at[wid], sidx_v)

        def g1(j, carry):
            gather(xhwc_hbm, sidx_v, xs_out, wid * C1W, j, rows_a, sem_a, True)
            gather(xhwc_hbm, sidx_v, xs_out, wid * C1W, j, rows_a, sem_a, False)
            return carry

        lax.fori_loop(0, G1, g1, 0)

    return body(table, xhwc, idx9, sidx)


def _k3_body(s_ref, e_ref, b_ref, p_ref, xs_ref, wk_ref, bias_ref, out_ref):
    i = pl.program_id(0)
    s = s_ref[i]
    e = e_ref[i]
    p = p_ref[...].astype(jnp.bfloat16)
    xs = xs_ref[...]
    bt = b_ref[0]

    def body(t, acc):
        w = wk_ref[t]
        y = jnp.dot(p, w, preferred_element_type=jnp.float32)
        y = y + bias_ref[t] + xs
        return jnp.where(bt == t, y, acc)

    out_ref[...] = lax.fori_loop(s, e + 1, body, jnp.zeros((MT, CC), jnp.float32))


def _k3(p_mat, xs, bsor3, wk, bias, s_arr, e_arr):
    grid_spec = pltpu.PrefetchScalarGridSpec(
        num_scalar_prefetch=2,
        grid=(NT,),
        in_specs=[
            pl.BlockSpec((1, MT, 1), lambda i, s, e: (i, 0, 0)),
            pl.BlockSpec((MT, 9 * CC), lambda i, s, e: (i, 0)),
            pl.BlockSpec((MT, CC), lambda i, s, e: (i, 0)),
            pl.BlockSpec((TT, 9 * CC, CC), lambda i, s, e: (0, 0, 0)),
            pl.BlockSpec((TT, 1, CC), lambda i, s, e: (0, 0, 0)),
        ],
        out_specs=pl.BlockSpec((MT, CC), lambda i, s, e: (i, 0)),
    )
    return pl.pallas_call(
        _k3_body,
        grid_spec=grid_spec,
        out_shape=jax.ShapeDtypeStruct((NN, CC), jnp.float32),
    )(s_arr, e_arr, bsor3, p_mat, xs, wk, bias)


def _k4(osort, sidx):
    mesh = plsc.VectorSubcoreMesh(core_axis_name="c", subcore_axis_name="s")

    @functools.partial(
        pl.kernel,
        out_type=jax.ShapeDtypeStruct((NN, CC), jnp.float32),
        mesh=mesh,
        scratch_types=[pltpu.VMEM((C1W, CHK), jnp.int32),
                       pltpu.VMEM((CHK, CC), jnp.float32),
                       pltpu.SemaphoreType.DMA],
        compiler_params=pltpu.CompilerParams(use_tc_tiling_on_sc=False),
    )
    def body(osort_hbm, sidx_hbm, out_hbm, sidx_v, buf_v, sem):
        wid = lax.axis_index("s") * 2 + lax.axis_index("c")
        pltpu.sync_copy(sidx_hbm.at[wid], sidx_v)

        def sc(j, carry):
            pltpu.sync_copy(
                osort_hbm.at[pl.ds((wid * C1W + j) * CHK, CHK)], buf_v)
            pltpu.async_copy(buf_v, out_hbm.at[sidx_v.at[j]], sem).wait()
            return carry

        lax.fori_loop(0, C1W, sc, 0)

    return body(osort, sidx)


def kernel(x, buckets, W1, b1, a1, W2, b2, a2, emb):
    f32 = jnp.float32
    bflat = buckets.reshape(NN).astype(jnp.int32)
    b_sorted, sort_idx = lax.sort_key_val(bflat, lax.iota(jnp.int32, NN))
    s_arr = b_sorted[0::MT]
    e_arr = b_sorted[MT - 1::MT]
    bsor3 = b_sorted.reshape(NT, MT, 1)

    h = sort_idx // WW
    w = sort_idx % WW
    offs = []
    for dy in (-1, 0, 1):
        rr = h + dy
        okr = (rr >= 0) & (rr < HH)
        for dx in (-1, 0, 1):
            offs.append(jnp.where(okr, rr * CP + (w + dx) + CO, 0))
    idx9 = jnp.stack(offs, axis=1).reshape(NW, C9W * CHK)
    sidx1 = sort_idx.reshape(NW, C1W * CHK)
    sidx3 = sort_idx.reshape(NW, C1W, CHK)

    w1m = W1.reshape(CC, CC, 9).transpose(2, 0, 1).astype(jnp.bfloat16)
    w2m = W2.reshape(CC, CC, 9).transpose(2, 1, 0).astype(jnp.bfloat16)
    aa = jnp.stack([a1[0], a2[0]])
    wk = emb[:, :CC * CC * 9].reshape(TT, CC, CC, 3, 3)
    wk = (0.1 * wk.transpose(0, 3, 4, 2, 1)).reshape(TT, 9 * CC, CC)
    wk = wk.astype(jnp.bfloat16)
    bias = (0.1 * emb[:, CC * CC * 9:]).reshape(TT, 1, CC)
    bb = jnp.stack([b1, b2])

    xq = jnp.pad(x[0], ((0, 0), (4, 4), (CO, CP - CO - WW)))
    xq = xq.reshape(CC, 392 * CP)
    table = _k1(xq, w1m, w2m, aa, bb)

    xhwc = x.reshape(CC, NN).T

    p_mat, xs = _k2(table, xhwc, idx9, sidx1)
    p_mat = p_mat.reshape(NN, 9 * CC)

    osort = _k3(p_mat, xs, bsor3, wk, bias, s_arr, e_arr)

    out_hwc = _k4(osort, sidx3)
    return out_hwc.T.reshape(1, CC, HH, WW)
